# baseline (speedup 1.0000x reference)
import functools

import jax
import jax.numpy as jnp
from jax import lax
from jax.experimental import pallas as pl
from jax.experimental.pallas import tpu as pltpu
from jax.experimental.pallas import tpu_sc as plsc

VOCAB = 1000
EMB = 16
B, L = 4096, 20
N = B * L

_NC, _NS = 2, 16
_NW = _NC * _NS
_ROWS_PER_W = N // _NW
_CHUNK = 64
_NCHUNK = _ROWS_PER_W // _CHUNK


def _proj_body(table_ref, w_ref, b_ref, out_ref):
    out_ref[...] = jnp.dot(
        table_ref[...], w_ref[...], preferred_element_type=jnp.float32
    ) + b_ref[...]


def _projected_table(table, W, b):
    return pl.pallas_call(
        _proj_body,
        out_shape=jax.ShapeDtypeStruct((VOCAB, VOCAB), jnp.float32),
    )(table, W, b.reshape(1, VOCAB))


_SC_MESH = plsc.VectorSubcoreMesh(core_axis_name="c", subcore_axis_name="s")


@functools.partial(
    pl.kernel,
    out_type=jax.ShapeDtypeStruct((N, VOCAB), jnp.float32),
    mesh=_SC_MESH,
    scratch_types=[
        pltpu.VMEM((_ROWS_PER_W,), jnp.int32),
        pltpu.VMEM((_CHUNK, VOCAB), jnp.float32),
        pltpu.SemaphoreType.DMA,
    ],
    compiler_params=pltpu.CompilerParams(use_tc_tiling_on_sc=False),
)
def _gather_rows(p_hbm, idx_hbm, out_hbm, idx_v, rows_v, sem):
    wid = lax.axis_index("s") * _NC + lax.axis_index("c")
    base = wid * _ROWS_PER_W
    pltpu.sync_copy(idx_hbm.at[pl.ds(base, _ROWS_PER_W)], idx_v)

    def body(ci, _):
        off = ci * _CHUNK
        pltpu.async_copy(
            p_hbm.at[idx_v.at[pl.ds(off, _CHUNK)]], rows_v, sem
        ).wait()
        pltpu.sync_copy(rows_v, out_hbm.at[pl.ds(base + off, _CHUNK)])
        return 0

    lax.fori_loop(0, _NCHUNK, body, 0)


def kernel(x, table, W, b):
    P = _projected_table(table, W, b)
    xflat = x.reshape(N).astype(jnp.int32)
    out = _gather_rows(P, xflat)
    return out.reshape(B, L, VOCAB)


# speedup vs baseline: 1.0136x; 1.0136x over previous
import functools

import jax
import jax.numpy as jnp
from jax import lax
from jax.experimental import pallas as pl
from jax.experimental.pallas import tpu as pltpu
from jax.experimental.pallas import tpu_sc as plsc

VOCAB = 1000
EMB = 16
B, L = 4096, 20
N = B * L

_NC, _NS = 2, 16
_NW = _NC * _NS
_ROWS_PER_W = N // _NW
_CHUNK = 64
_NCHUNK = _ROWS_PER_W // _CHUNK


def _proj_body(table_ref, w_ref, b_ref, out_ref):
    out_ref[...] = jnp.dot(
        table_ref[...], w_ref[...], preferred_element_type=jnp.float32
    ) + b_ref[...]


def _projected_table(table, W, b):
    return pl.pallas_call(
        _proj_body,
        out_shape=jax.ShapeDtypeStruct((VOCAB, VOCAB), jnp.float32),
    )(table, W, b.reshape(1, VOCAB))


_SC_MESH = plsc.VectorSubcoreMesh(core_axis_name="c", subcore_axis_name="s")


@functools.partial(
    pl.kernel,
    out_type=jax.ShapeDtypeStruct((N, VOCAB), jnp.float32),
    mesh=_SC_MESH,
    scratch_types=[
        pltpu.VMEM((_ROWS_PER_W,), jnp.int32),
        pltpu.VMEM((2 * _CHUNK, VOCAB), jnp.float32),
        pltpu.SemaphoreType.DMA,
        pltpu.SemaphoreType.DMA,
    ],
    compiler_params=pltpu.CompilerParams(use_tc_tiling_on_sc=False),
)
def _gather_rows(p_hbm, idx_hbm, out_hbm, idx_v, rows_v, gsem, wsem):
    wid = lax.axis_index("s") * _NC + lax.axis_index("c")
    base = wid * _ROWS_PER_W
    pltpu.sync_copy(idx_hbm.at[pl.ds(base, _ROWS_PER_W)], idx_v)

    def g_copy(ci, buf):
        return pltpu.make_async_copy(
            p_hbm.at[idx_v.at[pl.ds(ci * _CHUNK, _CHUNK)]],
            rows_v.at[pl.ds(buf * _CHUNK, _CHUNK)],
            gsem,
        )

    def w_copy(ci, buf):
        return pltpu.make_async_copy(
            rows_v.at[pl.ds(buf * _CHUNK, _CHUNK)],
            out_hbm.at[pl.ds(base + ci * _CHUNK, _CHUNK)],
            wsem,
        )

    g_copy(0, 0).start()

    def body(ci, _):
        buf = lax.rem(ci, 2)
        g_copy(ci, buf).wait()
        w_copy(ci, buf).start()

        @pl.when(ci >= 1)
        def _():
            w_copy(ci - 1, 1 - buf).wait()

        g_copy(ci + 1, 1 - buf).start()
        return 0

    lax.fori_loop(0, _NCHUNK - 1, body, 0)

    lastbuf = (_NCHUNK - 1) % 2
    g_copy(_NCHUNK - 1, lastbuf).wait()
    w_copy(_NCHUNK - 1, lastbuf).start()
    w_copy(_NCHUNK - 2, 1 - lastbuf).wait()
    w_copy(_NCHUNK - 1, lastbuf).wait()


def kernel(x, table, W, b):
    P = _projected_table(table, W, b)
    xflat = x.reshape(N).astype(jnp.int32)
    out = _gather_rows(P, xflat)
    return out.reshape(B, L, VOCAB)


# speedup vs baseline: 4.1088x; 4.0535x over previous
import functools

import jax
import jax.numpy as jnp
from jax import lax
from jax.experimental import pallas as pl
from jax.experimental.pallas import tpu as pltpu
from jax.experimental.pallas import tpu_sc as plsc

VOCAB = 1000
EMB = 16
B, L = 4096, 20
N = B * L

_NC, _NS = 2, 16
_NW = _NC * _NS
_B_PER_W = B // _NW

_SC_MESH = plsc.VectorSubcoreMesh(core_axis_name="c", subcore_axis_name="s")


@functools.partial(
    pl.kernel,
    out_type=jax.ShapeDtypeStruct((L, B, EMB), jnp.float32),
    mesh=_SC_MESH,
    scratch_types=[
        pltpu.VMEM((_B_PER_W,), jnp.int32),
        pltpu.VMEM((_B_PER_W, EMB), jnp.float32),
        pltpu.SemaphoreType.DMA,
    ],
    compiler_params=pltpu.CompilerParams(use_tc_tiling_on_sc=False),
)
def _sc_gather_emb(table_hbm, xt_hbm, emb_hbm, idx_v, rows_v, sem):
    wid = lax.axis_index("s") * _NC + lax.axis_index("c")
    b0 = wid * _B_PER_W

    def body(l, _):
        pltpu.sync_copy(xt_hbm.at[l, pl.ds(b0, _B_PER_W)], idx_v)
        pltpu.async_copy(table_hbm.at[idx_v], rows_v, sem).wait()
        pltpu.sync_copy(rows_v, emb_hbm.at[l, pl.ds(b0, _B_PER_W)])
        return 0

    lax.fori_loop(0, L, body, 0)


_BT = 1024


def _proj_body(w_ref, b_ref, emb_ref, out_ref):
    e = emb_ref[0]
    m = lax.dot_general(
        w_ref[...], e, (((0,), (1,)), ((), ())),
        preferred_element_type=jnp.float32,
    )
    out_ref[0] = m + b_ref[...]


def _tc_project(W, b2, emb3):
    grid = (L, B // _BT)
    return pl.pallas_call(
        _proj_body,
        grid=grid,
        in_specs=[
            pl.BlockSpec((EMB, VOCAB), lambda l, j: (0, 0)),
            pl.BlockSpec((VOCAB, 1), lambda l, j: (0, 0)),
            pl.BlockSpec((1, _BT, EMB), lambda l, j: (l, j, 0)),
        ],
        out_specs=pl.BlockSpec((1, VOCAB, _BT), lambda l, j: (l, 0, j)),
        out_shape=jax.ShapeDtypeStruct((L, VOCAB, B), jnp.float32),
    )(W, b2, emb3)


def kernel(x, table, W, b):
    xt = x.astype(jnp.int32).T
    emb3 = _sc_gather_emb(table, xt)
    outT = _tc_project(W, b.reshape(VOCAB, 1), emb3)
    return jnp.transpose(outT, (2, 0, 1))


# speedup vs baseline: 5.0506x; 1.2292x over previous
import functools

import jax
import jax.numpy as jnp
from jax import lax
from jax.experimental import pallas as pl
from jax.experimental.pallas import tpu as pltpu
from jax.experimental.pallas import tpu_sc as plsc

VOCAB = 1000
EMB = 16
B, L = 4096, 20
N = B * L

_NC, _NS = 2, 16
_NW = _NC * _NS
_B_PER_W = B // _NW

_SC_MESH = plsc.VectorSubcoreMesh(core_axis_name="c", subcore_axis_name="s")


@functools.partial(
    pl.kernel,
    out_type=jax.ShapeDtypeStruct((L, EMB, B), jnp.float32),
    mesh=_SC_MESH,
    scratch_types=[
        pltpu.VMEM((L, _B_PER_W), jnp.int32),
        pltpu.VMEM((VOCAB, EMB), jnp.float32),
        pltpu.VMEM((2, EMB, _B_PER_W), jnp.float32),
        pltpu.SemaphoreType.DMA,
    ],
    compiler_params=pltpu.CompilerParams(
        use_tc_tiling_on_sc=False, needs_layout_passes=False
    ),
)
def _sc_gather_emb(table_hbm, xt_hbm, emb_hbm, idx_v, tab_v, trans_v, wsem):
    wid = lax.axis_index("s") * _NC + lax.axis_index("c")
    b0 = wid * _B_PER_W
    pltpu.sync_copy(xt_hbm.at[:, pl.ds(b0, _B_PER_W)], idx_v)
    pltpu.sync_copy(table_hbm, tab_v)

    def w_copy(l, buf):
        return pltpu.make_async_copy(
            trans_v.at[buf],
            emb_hbm.at[l, :, pl.ds(b0, _B_PER_W)],
            wsem,
        )

    def body(li, _):
        for bb in range(2):
            l = li * 2 + bb

            @pl.when(l >= 2)
            def _():
                w_copy(l - 2, bb).wait()

            for k in range(_B_PER_W // 16):
                idxv = idx_v[l, pl.ds(k * 16, 16)]
                for e in range(EMB):
                    col = jnp.full((16,), e, jnp.int32)
                    vals = plsc.load_gather(tab_v, [idxv, col])
                    trans_v[bb, e, pl.ds(k * 16, 16)] = vals
            w_copy(l, bb).start()
        return 0

    lax.fori_loop(0, L // 2, body, 0)
    w_copy(L - 2, 0).wait()
    w_copy(L - 1, 1).wait()


_BT = 1024


def _proj_body(w_ref, b_ref, emb_ref, out_ref):
    e = emb_ref[0]
    m = lax.dot_general(
        w_ref[...], e, (((0,), (0,)), ((), ())),
        preferred_element_type=jnp.float32,
    )
    out_ref[0] = m + b_ref[...]


def _tc_project(W, b2, emb3):
    grid = (L, B // _BT)
    return pl.pallas_call(
        _proj_body,
        grid=grid,
        in_specs=[
            pl.BlockSpec((EMB, VOCAB), lambda l, j: (0, 0)),
            pl.BlockSpec((VOCAB, 1), lambda l, j: (0, 0)),
            pl.BlockSpec((1, EMB, _BT), lambda l, j: (l, 0, j)),
        ],
        out_specs=pl.BlockSpec((1, VOCAB, _BT), lambda l, j: (l, 0, j)),
        out_shape=jax.ShapeDtypeStruct((L, VOCAB, B), jnp.float32),
    )(W, b2, emb3)


def kernel(x, table, W, b):
    xt = x.astype(jnp.int32).T
    emb3 = _sc_gather_emb(table, xt)
    outT = _tc_project(W, b.reshape(VOCAB, 1), emb3)
    return jnp.transpose(outT, (2, 0, 1))
